# baseline (speedup 1.0000x reference)
import functools

import jax
import jax.numpy as jnp
from jax import lax
from jax.experimental import pallas as pl
from jax.experimental.pallas import tpu as pltpu
from jax.experimental.pallas import tpu_sc as plsc

B, T, V, E, H = 128, 32, 10000, 512, 512
G4 = 4 * H

NC, NS = 2, 16
NW = NC * NS
ROWS = B * T
RPW = ROWS // NW


def _gather_body(emb_hbm, idx_hbm, out_hbm, idx_v, rows_v, sem):
    wid = lax.axis_index("s") * NC + lax.axis_index("c")
    base = wid * RPW
    pltpu.sync_copy(idx_hbm.at[pl.ds(base, RPW)], idx_v)
    pltpu.async_copy(emb_hbm.at[idx_v], rows_v, sem).wait()
    pltpu.sync_copy(rows_v, out_hbm.at[pl.ds(base, RPW)])


def _sc_gather(emb, idx):
    mesh = plsc.VectorSubcoreMesh(
        core_axis_name="c", subcore_axis_name="s",
        num_cores=NC, num_subcores=NS)
    return pl.kernel(
        _gather_body,
        out_type=jax.ShapeDtypeStruct((ROWS, E), jnp.float32),
        mesh=mesh,
        scratch_types=[
            pltpu.VMEM((RPW,), jnp.int32),
            pltpu.VMEM((RPW, E), jnp.float32),
            pltpu.SemaphoreType.DMA,
        ],
    )(emb, idx)


def _lstm_body(x_ref, wi_ref, wh_ref, b_ref, h_out_ref, h_scr, c_scr,
               wi_bf, wh_bf):
    t = pl.program_id(0)

    @pl.when(t == 0)
    def _():
        h_scr[...] = jnp.zeros_like(h_scr)
        c_scr[...] = jnp.zeros_like(c_scr)
        wi_bf[...] = wi_ref[...].astype(jnp.bfloat16)
        wh_bf[...] = wh_ref[...].astype(jnp.bfloat16)

    x = x_ref[0].astype(jnp.bfloat16)
    h = h_scr[...]
    g = (jnp.dot(x, wi_bf[...], preferred_element_type=jnp.float32)
         + jnp.dot(h.astype(jnp.bfloat16), wh_bf[...],
                   preferred_element_type=jnp.float32)
         + b_ref[...])
    gi = jax.nn.sigmoid(g[:, 0:H])
    gf = jax.nn.sigmoid(g[:, H:2 * H])
    gg = jnp.tanh(g[:, 2 * H:3 * H])
    go = jax.nn.sigmoid(g[:, 3 * H:4 * H])
    c = gf * c_scr[...] + gi * gg
    h = go * jnp.tanh(c)
    c_scr[...] = c
    h_scr[...] = h

    @pl.when(t == T - 1)
    def _():
        h_out_ref[...] = h


def _lstm(x, Wi, Wh, b):
    return pl.pallas_call(
        _lstm_body,
        grid=(T,),
        in_specs=[
            pl.BlockSpec((1, B, E), lambda t: (t, 0, 0)),
            pl.BlockSpec((E, G4), lambda t: (0, 0)),
            pl.BlockSpec((H, G4), lambda t: (0, 0)),
            pl.BlockSpec((1, G4), lambda t: (0, 0)),
        ],
        out_specs=pl.BlockSpec((B, H), lambda t: (0, 0)),
        out_shape=jax.ShapeDtypeStruct((B, H), jnp.float32),
        scratch_shapes=[
            pltpu.VMEM((B, H), jnp.float32),
            pltpu.VMEM((B, H), jnp.float32),
            pltpu.VMEM((E, G4), jnp.bfloat16),
            pltpu.VMEM((H, G4), jnp.bfloat16),
        ],
    )(x, Wi, Wh, b)


BV = 2048


def _proj_body(h_ref, w_ref, b_ref, out_ref):
    out_ref[...] = (jnp.dot(h_ref[...].astype(jnp.bfloat16),
                            w_ref[...].astype(jnp.bfloat16),
                            preferred_element_type=jnp.float32)
                    + b_ref[...])


def _proj(h, W_out, b_out):
    nv = pl.cdiv(V, BV)
    return pl.pallas_call(
        _proj_body,
        grid=(nv,),
        in_specs=[
            pl.BlockSpec((B, H), lambda v: (0, 0)),
            pl.BlockSpec((H, BV), lambda v: (0, v)),
            pl.BlockSpec((1, BV), lambda v: (0, v)),
        ],
        out_specs=pl.BlockSpec((B, BV), lambda v: (0, v)),
        out_shape=jax.ShapeDtypeStruct((B, V), jnp.float32),
    )(h, W_out, b_out)


def kernel(surf, emb, Wi, Wh, b, W_root, b_root, cb_root, W_ord, b_ord,
           cb_ord, W_out, b_out):
    idx = jnp.transpose(surf).reshape(ROWS).astype(jnp.int32)
    x = _sc_gather(emb, idx).reshape(T, B, E)
    h = _lstm(x, Wi, Wh, b.reshape(1, G4))
    logits = _proj(h, W_out, b_out.reshape(1, V))
    return logits.reshape(B, 1, V)


# speedup vs baseline: 1.0243x; 1.0243x over previous
import functools

import jax
import jax.numpy as jnp
from jax import lax
from jax.experimental import pallas as pl
from jax.experimental.pallas import tpu as pltpu
from jax.experimental.pallas import tpu_sc as plsc

B, T, V, E, H = 128, 32, 10000, 512, 512
G4 = 4 * H

NC, NS = 2, 16
NW = NC * NS
ROWS = B * T
RPW = ROWS // NW


def _gather_body(emb_hbm, idx_hbm, out_hbm, idx_v, rows_v, sem):
    wid = lax.axis_index("s") * NC + lax.axis_index("c")
    base = wid * RPW
    pltpu.sync_copy(idx_hbm.at[pl.ds(base, RPW)], idx_v)
    pltpu.async_copy(emb_hbm.at[idx_v], rows_v, sem).wait()
    pltpu.sync_copy(rows_v, out_hbm.at[pl.ds(base, RPW)])


def _sc_gather(emb, idx):
    mesh = plsc.VectorSubcoreMesh(
        core_axis_name="c", subcore_axis_name="s",
        num_cores=NC, num_subcores=NS)
    return pl.kernel(
        _gather_body,
        out_type=jax.ShapeDtypeStruct((ROWS, E), jnp.float32),
        mesh=mesh,
        scratch_types=[
            pltpu.VMEM((RPW,), jnp.int32),
            pltpu.VMEM((RPW, E), jnp.float32),
            pltpu.SemaphoreType.DMA,
        ],
    )(emb, idx)


BV = 2048
NVBLK = 5


def _fused_body(x_ref, wi_ref, wh_ref, b_ref, bout_ref, wout_hbm,
                out_ref, h_scr, c_scr, wi_bf, wh_bf, wout_vmem, dma_sem):
    t = pl.program_id(0)

    @pl.when(t == 0)
    def _():
        h_scr[...] = jnp.zeros_like(h_scr)
        c_scr[...] = jnp.zeros_like(c_scr)
        wi_bf[...] = wi_ref[...].astype(jnp.bfloat16)
        wh_bf[...] = wh_ref[...].astype(jnp.bfloat16)
        pltpu.make_async_copy(wout_hbm, wout_vmem, dma_sem).start()

    x = x_ref[0].astype(jnp.bfloat16)
    h = h_scr[...]
    g = (jnp.dot(x, wi_bf[...], preferred_element_type=jnp.float32)
         + jnp.dot(h.astype(jnp.bfloat16), wh_bf[...],
                   preferred_element_type=jnp.float32)
         + b_ref[...])
    gi = jax.nn.sigmoid(g[:, 0:H])
    gf = jax.nn.sigmoid(g[:, H:2 * H])
    gg = jnp.tanh(g[:, 2 * H:3 * H])
    go = jax.nn.sigmoid(g[:, 3 * H:4 * H])
    c = gf * c_scr[...] + gi * gg
    h = go * jnp.tanh(c)
    c_scr[...] = c
    h_scr[...] = h

    @pl.when(t == T - 1)
    def _():
        pltpu.make_async_copy(wout_hbm, wout_vmem, dma_sem).wait()
        hb = h.astype(jnp.bfloat16)
        for j in range(NVBLK):
            lo = j * BV
            w = min(BV, V - lo)
            wblk = wout_vmem[:, lo:lo + w].astype(jnp.bfloat16)
            out_ref[:, lo:lo + w] = (
                jnp.dot(hb, wblk, preferred_element_type=jnp.float32)
                + bout_ref[:, lo:lo + w])


def _fused(x, Wi, Wh, b, W_out, b_out):
    return pl.pallas_call(
        _fused_body,
        grid=(T,),
        in_specs=[
            pl.BlockSpec((1, B, E), lambda t: (t, 0, 0)),
            pl.BlockSpec((E, G4), lambda t: (0, 0)),
            pl.BlockSpec((H, G4), lambda t: (0, 0)),
            pl.BlockSpec((1, G4), lambda t: (0, 0)),
            pl.BlockSpec((1, V), lambda t: (0, 0)),
            pl.BlockSpec(memory_space=pltpu.HBM),
        ],
        out_specs=pl.BlockSpec((B, V), lambda t: (0, 0)),
        out_shape=jax.ShapeDtypeStruct((B, V), jnp.float32),
        scratch_shapes=[
            pltpu.VMEM((B, H), jnp.float32),
            pltpu.VMEM((B, H), jnp.float32),
            pltpu.VMEM((E, G4), jnp.bfloat16),
            pltpu.VMEM((H, G4), jnp.bfloat16),
            pltpu.VMEM((H, V), jnp.float32),
            pltpu.SemaphoreType.DMA,
        ],
    )(x, Wi, Wh, b, b_out, W_out)


def kernel(surf, emb, Wi, Wh, b, W_root, b_root, cb_root, W_ord, b_ord,
           cb_ord, W_out, b_out):
    idx = jnp.transpose(surf).reshape(ROWS).astype(jnp.int32)
    x = _sc_gather(emb, idx).reshape(T, B, E)
    logits = _fused(x, Wi, Wh, b.reshape(1, G4), W_out, b_out.reshape(1, V))
    return logits.reshape(B, 1, V)


# speedup vs baseline: 1.3002x; 1.2693x over previous
import functools

import jax
import jax.numpy as jnp
from jax import lax
from jax.experimental import pallas as pl
from jax.experimental.pallas import tpu as pltpu
from jax.experimental.pallas import tpu_sc as plsc

B, T, V, E, H = 128, 32, 10000, 512, 512
G4 = 4 * H

NC, NS = 2, 16
NW = NC * NS
ROWS = B * T
RPW = ROWS // NW


def _gather_body(emb_hbm, idx_hbm, out_hbm, idx_v, rows_v, sem):
    wid = lax.axis_index("s") * NC + lax.axis_index("c")
    base = wid * RPW
    pltpu.sync_copy(idx_hbm.at[pl.ds(base, RPW)], idx_v)
    pltpu.async_copy(emb_hbm.at[idx_v], rows_v, sem).wait()
    pltpu.sync_copy(rows_v, out_hbm.at[pl.ds(base, RPW)])


def _sc_gather(emb, idx):
    mesh = plsc.VectorSubcoreMesh(
        core_axis_name="c", subcore_axis_name="s",
        num_cores=NC, num_subcores=NS)
    return pl.kernel(
        _gather_body,
        out_type=jax.ShapeDtypeStruct((ROWS, E), jnp.float32),
        mesh=mesh,
        scratch_types=[
            pltpu.VMEM((RPW,), jnp.int32),
            pltpu.VMEM((RPW, E), jnp.float32),
            pltpu.SemaphoreType.DMA,
        ],
    )(emb, idx)


BV = 2048
NVBLK = 5


def _fused_body(x_ref, wi_ref, wh_ref, b_ref, bout_ref, wout_hbm,
                out_ref, h_scr, c_scr, wi_bf, wh_bf, wout_vmem, dma_sem):
    t = pl.program_id(0)

    @pl.when(t == 0)
    def _():
        h_scr[...] = jnp.zeros_like(h_scr)
        c_scr[...] = jnp.zeros_like(c_scr)
        wi_bf[...] = wi_ref[...].astype(jnp.bfloat16)
        wh_bf[...] = wh_ref[...].astype(jnp.bfloat16)
        pltpu.make_async_copy(wout_hbm, wout_vmem, dma_sem).start()

    x = x_ref[0].astype(jnp.bfloat16)
    h = h_scr[...]
    g = (jnp.dot(x, wi_bf[...], preferred_element_type=jnp.float32)
         + jnp.dot(h.astype(jnp.bfloat16), wh_bf[...],
                   preferred_element_type=jnp.float32)
         + b_ref[...])
    gi = jax.nn.sigmoid(g[:, 0:H])
    gf = jax.nn.sigmoid(g[:, H:2 * H])
    gg = jnp.tanh(g[:, 2 * H:3 * H])
    go = jax.nn.sigmoid(g[:, 3 * H:4 * H])
    c = gf * c_scr[...] + gi * gg
    h = go * jnp.tanh(c)
    c_scr[...] = c
    h_scr[...] = h

    @pl.when(t == T - 1)
    def _():
        pltpu.make_async_copy(wout_hbm, wout_vmem, dma_sem).wait()
        hb = h.astype(jnp.bfloat16)
        for j in range(NVBLK):
            lo = j * BV
            w = min(BV, V - lo)
            wblk = wout_vmem[lo:lo + w, :].astype(jnp.bfloat16)
            acc = jax.lax.dot_general(
                wblk, hb, (((1,), (1,)), ((), ())),
                preferred_element_type=jnp.float32)
            col = bout_ref[0, lo:lo + w].reshape(w, 1)
            out_ref[lo:lo + w, :] = acc + col


def _fused(x, Wi, Wh, b, W_out_t, b_out):
    return pl.pallas_call(
        _fused_body,
        grid=(T,),
        in_specs=[
            pl.BlockSpec((1, B, E), lambda t: (t, 0, 0)),
            pl.BlockSpec((E, G4), lambda t: (0, 0)),
            pl.BlockSpec((H, G4), lambda t: (0, 0)),
            pl.BlockSpec((1, G4), lambda t: (0, 0)),
            pl.BlockSpec((1, V), lambda t: (0, 0)),
            pl.BlockSpec(memory_space=pltpu.HBM),
        ],
        out_specs=pl.BlockSpec((V, B), lambda t: (0, 0)),
        out_shape=jax.ShapeDtypeStruct((V, B), jnp.float32),
        scratch_shapes=[
            pltpu.VMEM((B, H), jnp.float32),
            pltpu.VMEM((B, H), jnp.float32),
            pltpu.VMEM((E, G4), jnp.bfloat16),
            pltpu.VMEM((H, G4), jnp.bfloat16),
            pltpu.VMEM((V, H), jnp.float32),
            pltpu.SemaphoreType.DMA,
        ],
    )(x, Wi, Wh, b, b_out, W_out_t)


def kernel(surf, emb, Wi, Wh, b, W_root, b_root, cb_root, W_ord, b_ord,
           cb_ord, W_out, b_out):
    idx = jnp.transpose(surf).reshape(ROWS).astype(jnp.int32)
    x = _sc_gather(emb, idx).reshape(T, B, E)
    logits_t = _fused(x, Wi, Wh, b.reshape(1, G4), jnp.transpose(W_out),
                      b_out.reshape(1, V))
    return jnp.transpose(logits_t).reshape(B, 1, V)
